# Initial kernel scaffold; baseline (speedup 1.0000x reference)
import jax
import jax.numpy as jnp
from jax.experimental import pallas as pl


def kernel(img1, img2):
    raise NotImplementedError("write your pallas kernel here")


# speedup vs baseline: 273.9144x; 273.9144x over previous
import numpy as np

import jax
import jax.numpy as jnp
from jax.experimental import pallas as pl
from jax.experimental.pallas import tpu as pltpu

_WINDOW_SIZE = 11
_SIGMA = 1.5
_C1 = 0.01 ** 2
_C2 = 0.03 ** 2


def _gauss_taps():
    x = np.arange(_WINDOW_SIZE) - _WINDOW_SIZE // 2
    g = np.exp(-(x ** 2) / (2.0 * _SIGMA ** 2))
    return g / g.sum()


def _band_matrix(n, dtype):
    g = _gauss_taps()
    m = np.zeros((n, n), dtype=np.float64)
    for t in range(-(_WINDOW_SIZE // 2), _WINDOW_SIZE // 2 + 1):
        d = np.full(n - abs(t), g[t + _WINDOW_SIZE // 2])
        m += np.diag(d, k=t)
    return jnp.asarray(m, dtype=dtype)


def _bf16_tap_renorm():
    import ml_dtypes
    taps_bf16 = _gauss_taps().astype(ml_dtypes.bfloat16)
    s = float(np.sum(np.float64(taps_bf16)))
    return np.float32(1.0 / (s * s))


def _ssim_body(x_ref, y_ref, mv_ref, mh_ref, o_ref):
    x = x_ref[0]
    y = y_ref[0]
    mv = mv_ref[...]
    mh = mh_ref[...]
    renorm = _bf16_tap_renorm()

    def blur(q):
        h = jnp.dot(q.astype(jnp.bfloat16), mh,
                    preferred_element_type=jnp.float32)
        return jnp.dot(mv, h.astype(jnp.bfloat16),
                       preferred_element_type=jnp.float32) * renorm

    mu1 = blur(x)
    mu2 = blur(y)
    bxx = blur(x * x)
    byy = blur(y * y)
    bxy = blur(x * y)

    mu1_sq = mu1 * mu1
    mu2_sq = mu2 * mu2
    mu1_mu2 = mu1 * mu2
    sigma1_sq = bxx - mu1_sq
    sigma2_sq = byy - mu2_sq
    sigma12 = bxy - mu1_mu2

    num = (2.0 * mu1_mu2 + _C1) * (2.0 * sigma12 + _C2)
    den = (mu1_sq + mu2_sq + _C1) * (sigma1_sq + sigma2_sq + _C2)
    ssim_map = num / den
    o_ref[0] = jnp.sum(ssim_map, keepdims=True)


def kernel(img1, img2):
    b, c, h, w = img1.shape
    n = b * c
    x = img1.reshape(n, h, w)
    y = img2.reshape(n, h, w)
    mv = _band_matrix(h, jnp.bfloat16)
    mh = mv if h == w else _band_matrix(w, jnp.bfloat16)

    sums = pl.pallas_call(
        _ssim_body,
        grid=(n,),
        in_specs=[
            pl.BlockSpec((1, h, w), lambda i: (i, 0, 0)),
            pl.BlockSpec((1, h, w), lambda i: (i, 0, 0)),
            pl.BlockSpec((h, h), lambda i: (0, 0)),
            pl.BlockSpec((w, w), lambda i: (0, 0)),
        ],
        out_specs=pl.BlockSpec((1, 1, 1), lambda i: (i, 0, 0)),
        out_shape=jax.ShapeDtypeStruct((n, 1, 1), jnp.float32),
        compiler_params=pltpu.CompilerParams(
            dimension_semantics=("parallel",),
        ),
    )(x, y, mv, mh)

    return 1.0 - jnp.sum(sums) / (n * h * w)


# speedup vs baseline: 349.7920x; 1.2770x over previous
import ml_dtypes
import numpy as np

import jax
import jax.numpy as jnp
from jax.experimental import pallas as pl
from jax.experimental.pallas import tpu as pltpu

_WINDOW_SIZE = 11
_SIGMA = 1.5
_C1 = 0.01 ** 2
_C2 = 0.03 ** 2


def _gauss_taps():
    x = np.arange(_WINDOW_SIZE) - _WINDOW_SIZE // 2
    g = np.exp(-(x ** 2) / (2.0 * _SIGMA ** 2))
    return g / g.sum()


def _taps_bf16_exact_sum():
    b = _gauss_taps().astype(ml_dtypes.bfloat16)
    b64 = np.float64(b)
    ulp = np.float64(np.nextafter(b, np.ones_like(b), dtype=ml_dtypes.bfloat16)) - b64
    residue = 1.0 - b64.sum()
    order = np.argsort(-ulp)
    for i in order:
        k = np.round(residue / ulp[i])
        k = np.clip(k, -3, 3)
        if k != 0.0:
            b64[i] = b64[i] + k * ulp[i]
            residue = residue - k * ulp[i]
    assert residue == 0.0 and b64.sum() == 1.0, residue
    assert np.all(np.float64(b64.astype(ml_dtypes.bfloat16)) == b64)
    return b64


def _band_matrix(n):
    g = _taps_bf16_exact_sum()
    m = np.zeros((n, n), dtype=np.float64)
    for t in range(-(_WINDOW_SIZE // 2), _WINDOW_SIZE // 2 + 1):
        d = np.full(n - abs(t), g[t + _WINDOW_SIZE // 2])
        m += np.diag(d, k=t)
    return jnp.asarray(m, dtype=jnp.bfloat16)


_N_STRIPS = 2


def _ssim_body(x_ref, y_ref, m_ref, o_ref, hb_ref):
    m = m_ref[...]
    bf = jnp.bfloat16
    f32 = jnp.float32
    g_batch, h, w = x_ref.shape

    for g in range(g_batch):
        x = x_ref[g]
        y = y_ref[g]

        xb = x.astype(bf)
        yb = y.astype(bf)
        stack = jnp.concatenate([xb, yb, xb * xb, yb * yb, xb * yb], axis=0)
        hb_ref[g] = jnp.dot(stack, m, preferred_element_type=f32).astype(bf)

        sw = w // _N_STRIPS
        acc = None
        for s in range(_N_STRIPS):
            c0 = s * sw
            mu1, mu2, bxx, byy, bxy = (
                jnp.dot(m, hb_ref[g, q * h:(q + 1) * h, c0:c0 + sw],
                        preferred_element_type=f32)
                for q in range(5))
            mu1_sq = mu1 * mu1
            mu2_sq = mu2 * mu2
            mu1_mu2 = mu1 * mu2
            sigma1_sq = bxx - mu1_sq
            sigma2_sq = byy - mu2_sq
            sigma12 = bxy - mu1_mu2
            num = (2.0 * mu1_mu2 + _C1) * (2.0 * sigma12 + _C2)
            den = (mu1_sq + mu2_sq + _C1) * (sigma1_sq + sigma2_sq + _C2)
            part = jnp.sum(num / den, keepdims=True)
            acc = part if acc is None else acc + part
        o_ref[g] = acc


def kernel(img1, img2):
    b, c, h, w = img1.shape
    n = b * c
    x = img1.reshape(n, h, w)
    y = img2.reshape(n, h, w)
    assert h == w, "band matrix is shared between the H and V passes"
    m = _band_matrix(h)

    g_batch = 2
    sums = pl.pallas_call(
        _ssim_body,
        grid=(n // g_batch,),
        in_specs=[
            pl.BlockSpec((g_batch, h, w), lambda i: (i, 0, 0)),
            pl.BlockSpec((g_batch, h, w), lambda i: (i, 0, 0)),
            pl.BlockSpec((h, h), lambda i: (0, 0)),
        ],
        out_specs=pl.BlockSpec((g_batch, 1, 1), lambda i: (i, 0, 0)),
        out_shape=jax.ShapeDtypeStruct((n, 1, 1), jnp.float32),
        scratch_shapes=[pltpu.VMEM((g_batch, 5 * h, w), jnp.bfloat16)],
        compiler_params=pltpu.CompilerParams(
            dimension_semantics=("arbitrary",),
        ),
    )(x, y, m)

    return 1.0 - jnp.sum(sums) / (n * h * w)


# speedup vs baseline: 353.6323x; 1.0110x over previous
import ml_dtypes
import numpy as np

import jax
import jax.numpy as jnp
from jax.experimental import pallas as pl
from jax.experimental.pallas import tpu as pltpu

_WINDOW_SIZE = 11
_SIGMA = 1.5
_C1 = 0.01 ** 2
_C2 = 0.03 ** 2


def _gauss_taps():
    x = np.arange(_WINDOW_SIZE) - _WINDOW_SIZE // 2
    g = np.exp(-(x ** 2) / (2.0 * _SIGMA ** 2))
    return g / g.sum()


def _taps_bf16_exact_sum():
    b = _gauss_taps().astype(ml_dtypes.bfloat16)
    b64 = np.float64(b)
    ulp = np.float64(np.nextafter(b, np.ones_like(b), dtype=ml_dtypes.bfloat16)) - b64
    residue = 1.0 - b64.sum()
    order = np.argsort(-ulp)
    for i in order:
        k = np.round(residue / ulp[i])
        k = np.clip(k, -3, 3)
        if k != 0.0:
            b64[i] = b64[i] + k * ulp[i]
            residue = residue - k * ulp[i]
    assert residue == 0.0 and b64.sum() == 1.0, residue
    assert np.all(np.float64(b64.astype(ml_dtypes.bfloat16)) == b64)
    return b64


def _band_matrix(n):
    g = _taps_bf16_exact_sum()
    m = np.zeros((n, n), dtype=np.float64)
    for t in range(-(_WINDOW_SIZE // 2), _WINDOW_SIZE // 2 + 1):
        d = np.full(n - abs(t), g[t + _WINDOW_SIZE // 2])
        m += np.diag(d, k=t)
    return jnp.asarray(m, dtype=jnp.bfloat16)


_N_STRIPS = 2


def _ssim_body(x_ref, y_ref, m_ref, o_ref, hb_ref):
    m = m_ref[...]
    bf = jnp.bfloat16
    f32 = jnp.float32
    g_batch, h, w = x_ref.shape

    for g in range(g_batch):
        x = x_ref[g]
        y = y_ref[g]

        xb = x.astype(bf)
        yb = y.astype(bf)
        stack = jnp.concatenate([xb, yb, xb * xb, yb * yb, xb * yb], axis=0)
        hb_ref[g] = jnp.dot(stack, m, preferred_element_type=f32).astype(bf)

        sw = w // _N_STRIPS
        acc = None
        for s in range(_N_STRIPS):
            c0 = s * sw
            mu1, mu2, bxx, byy, bxy = (
                jnp.dot(m, hb_ref[g, q * h:(q + 1) * h, c0:c0 + sw],
                        preferred_element_type=f32)
                for q in range(5))
            mu1_sq = mu1 * mu1
            mu2_sq = mu2 * mu2
            mu1_mu2 = mu1 * mu2
            sigma1_sq = bxx - mu1_sq
            sigma2_sq = byy - mu2_sq
            sigma12 = bxy - mu1_mu2
            num = (2.0 * mu1_mu2 + _C1) * (2.0 * sigma12 + _C2)
            den = (mu1_sq + mu2_sq + _C1) * (sigma1_sq + sigma2_sq + _C2)
            part = jnp.sum(num / den, keepdims=True)
            acc = part if acc is None else acc + part
        o_ref[g] = acc


def kernel(img1, img2):
    b, c, h, w = img1.shape
    n = b * c
    x = img1.reshape(n, h, w)
    y = img2.reshape(n, h, w)
    assert h == w, "band matrix is shared between the H and V passes"
    m = _band_matrix(h)

    g_batch = 4
    sums = pl.pallas_call(
        _ssim_body,
        grid=(n // g_batch,),
        in_specs=[
            pl.BlockSpec((g_batch, h, w), lambda i: (i, 0, 0)),
            pl.BlockSpec((g_batch, h, w), lambda i: (i, 0, 0)),
            pl.BlockSpec((h, h), lambda i: (0, 0)),
        ],
        out_specs=pl.BlockSpec((g_batch, 1, 1), lambda i: (i, 0, 0)),
        out_shape=jax.ShapeDtypeStruct((n, 1, 1), jnp.float32),
        scratch_shapes=[pltpu.VMEM((g_batch, 5 * h, w), jnp.bfloat16)],
        compiler_params=pltpu.CompilerParams(
            dimension_semantics=("arbitrary",),
        ),
    )(x, y, m)

    return 1.0 - jnp.sum(sums) / (n * h * w)


# speedup vs baseline: 357.9156x; 1.0121x over previous
import ml_dtypes
import numpy as np

import jax
import jax.numpy as jnp
from jax.experimental import pallas as pl
from jax.experimental.pallas import tpu as pltpu

_WINDOW_SIZE = 11
_SIGMA = 1.5
_C1 = 0.01 ** 2
_C2 = 0.03 ** 2


def _gauss_taps():
    x = np.arange(_WINDOW_SIZE) - _WINDOW_SIZE // 2
    g = np.exp(-(x ** 2) / (2.0 * _SIGMA ** 2))
    return g / g.sum()


def _taps_bf16_exact_sum():
    b = _gauss_taps().astype(ml_dtypes.bfloat16)
    b64 = np.float64(b)
    ulp = np.float64(np.nextafter(b, np.ones_like(b), dtype=ml_dtypes.bfloat16)) - b64
    residue = 1.0 - b64.sum()
    order = np.argsort(-ulp)
    for i in order:
        k = np.round(residue / ulp[i])
        k = np.clip(k, -3, 3)
        if k != 0.0:
            b64[i] = b64[i] + k * ulp[i]
            residue = residue - k * ulp[i]
    assert residue == 0.0 and b64.sum() == 1.0, residue
    assert np.all(np.float64(b64.astype(ml_dtypes.bfloat16)) == b64)
    return b64


def _band_matrix(n):
    g = _taps_bf16_exact_sum()
    m = np.zeros((n, n), dtype=np.float64)
    for t in range(-(_WINDOW_SIZE // 2), _WINDOW_SIZE // 2 + 1):
        d = np.full(n - abs(t), g[t + _WINDOW_SIZE // 2])
        m += np.diag(d, k=t)
    return jnp.asarray(m, dtype=jnp.bfloat16)


_N_STRIPS = 2


def _ssim_body(x_ref, y_ref, m_ref, o_ref, hb_ref):
    m = m_ref[...]
    bf = jnp.bfloat16
    f32 = jnp.float32
    g_batch, h, w = x_ref.shape

    stacks = []
    for g in range(g_batch):
        xb = x_ref[g].astype(bf)
        yb = y_ref[g].astype(bf)
        stacks += [xb, yb, xb * xb, yb * yb, xb * yb]
    big = jnp.dot(jnp.concatenate(stacks, axis=0), m,
                  preferred_element_type=f32).astype(bf)
    hb_ref[...] = big.reshape(g_batch, 5 * h, w)

    for g in range(g_batch):
        sw = w // _N_STRIPS
        acc = None
        for s in range(_N_STRIPS):
            c0 = s * sw
            mu1, mu2, bxx, byy, bxy = (
                jnp.dot(m, hb_ref[g, q * h:(q + 1) * h, c0:c0 + sw],
                        preferred_element_type=f32)
                for q in range(5))
            mu1_sq = mu1 * mu1
            mu2_sq = mu2 * mu2
            mu1_mu2 = mu1 * mu2
            sigma1_sq = bxx - mu1_sq
            sigma2_sq = byy - mu2_sq
            sigma12 = bxy - mu1_mu2
            num = (2.0 * mu1_mu2 + _C1) * (2.0 * sigma12 + _C2)
            den = (mu1_sq + mu2_sq + _C1) * (sigma1_sq + sigma2_sq + _C2)
            part = jnp.sum(num / den, keepdims=True)
            acc = part if acc is None else acc + part
        o_ref[g] = acc


def kernel(img1, img2):
    b, c, h, w = img1.shape
    n = b * c
    x = img1.reshape(n, h, w)
    y = img2.reshape(n, h, w)
    assert h == w, "band matrix is shared between the H and V passes"
    m = _band_matrix(h)

    g_batch = 4
    sums = pl.pallas_call(
        _ssim_body,
        grid=(n // g_batch,),
        in_specs=[
            pl.BlockSpec((g_batch, h, w), lambda i: (i, 0, 0)),
            pl.BlockSpec((g_batch, h, w), lambda i: (i, 0, 0)),
            pl.BlockSpec((h, h), lambda i: (0, 0)),
        ],
        out_specs=pl.BlockSpec((g_batch, 1, 1), lambda i: (i, 0, 0)),
        out_shape=jax.ShapeDtypeStruct((n, 1, 1), jnp.float32),
        scratch_shapes=[pltpu.VMEM((g_batch, 5 * h, w), jnp.bfloat16)],
        compiler_params=pltpu.CompilerParams(
            dimension_semantics=("arbitrary",),
        ),
    )(x, y, m)

    return 1.0 - jnp.sum(sums) / (n * h * w)


# speedup vs baseline: 358.8857x; 1.0027x over previous
import ml_dtypes
import numpy as np

import jax
import jax.numpy as jnp
from jax.experimental import pallas as pl
from jax.experimental.pallas import tpu as pltpu

_WINDOW_SIZE = 11
_SIGMA = 1.5
_C1 = 0.01 ** 2
_C2 = 0.03 ** 2


def _gauss_taps():
    x = np.arange(_WINDOW_SIZE) - _WINDOW_SIZE // 2
    g = np.exp(-(x ** 2) / (2.0 * _SIGMA ** 2))
    return g / g.sum()


def _taps_bf16_exact_sum():
    b = _gauss_taps().astype(ml_dtypes.bfloat16)
    b64 = np.float64(b)
    ulp = np.float64(np.nextafter(b, np.ones_like(b), dtype=ml_dtypes.bfloat16)) - b64
    residue = 1.0 - b64.sum()
    order = np.argsort(-ulp)
    for i in order:
        k = np.round(residue / ulp[i])
        k = np.clip(k, -3, 3)
        if k != 0.0:
            b64[i] = b64[i] + k * ulp[i]
            residue = residue - k * ulp[i]
    assert residue == 0.0 and b64.sum() == 1.0, residue
    assert np.all(np.float64(b64.astype(ml_dtypes.bfloat16)) == b64)
    return b64


def _band_matrix(n):
    g = _taps_bf16_exact_sum()
    m = np.zeros((n, n), dtype=np.float64)
    for t in range(-(_WINDOW_SIZE // 2), _WINDOW_SIZE // 2 + 1):
        d = np.full(n - abs(t), g[t + _WINDOW_SIZE // 2])
        m += np.diag(d, k=t)
    return jnp.asarray(m, dtype=jnp.bfloat16)


_N_STRIPS = 2


def _ssim_body(x_ref, y_ref, m_ref, o_ref, hb_ref):
    m = m_ref[...]
    bf = jnp.bfloat16
    f32 = jnp.float32
    g_batch, h, w = x_ref.shape

    stacks = []
    for g in range(g_batch):
        xb = x_ref[g].astype(bf)
        yb = y_ref[g].astype(bf)
        stacks += [xb, yb, xb * xb, yb * yb, xb * yb]
    big = jnp.dot(jnp.concatenate(stacks, axis=0), m,
                  preferred_element_type=f32).astype(bf)
    hb_ref[...] = big.reshape(g_batch, 5 * h, w)

    for g in range(g_batch):
        sw = w // _N_STRIPS
        acc = None
        for s in range(_N_STRIPS):
            c0 = s * sw
            mu1, mu2, bxx, byy, bxy = (
                jnp.dot(m, hb_ref[g, q * h:(q + 1) * h, c0:c0 + sw],
                        preferred_element_type=f32)
                for q in range(5))
            mu1_sq = mu1 * mu1
            mu2_sq = mu2 * mu2
            mu1_mu2 = mu1 * mu2
            sigma1_sq = bxx - mu1_sq
            sigma2_sq = byy - mu2_sq
            sigma12 = bxy - mu1_mu2
            num = (2.0 * mu1_mu2 + _C1) * (2.0 * sigma12 + _C2)
            den = (mu1_sq + mu2_sq + _C1) * (sigma1_sq + sigma2_sq + _C2)
            part = jnp.sum(num / den, keepdims=True)
            acc = part if acc is None else acc + part
        o_ref[g] = acc


def kernel(img1, img2):
    b, c, h, w = img1.shape
    n = b * c
    x = img1.reshape(n, h, w)
    y = img2.reshape(n, h, w)
    assert h == w, "band matrix is shared between the H and V passes"
    m = _band_matrix(h)

    g_batch = 6
    sums = pl.pallas_call(
        _ssim_body,
        grid=(n // g_batch,),
        in_specs=[
            pl.BlockSpec((g_batch, h, w), lambda i: (i, 0, 0)),
            pl.BlockSpec((g_batch, h, w), lambda i: (i, 0, 0)),
            pl.BlockSpec((h, h), lambda i: (0, 0)),
        ],
        out_specs=pl.BlockSpec((g_batch, 1, 1), lambda i: (i, 0, 0)),
        out_shape=jax.ShapeDtypeStruct((n, 1, 1), jnp.float32),
        scratch_shapes=[pltpu.VMEM((g_batch, 5 * h, w), jnp.bfloat16)],
        compiler_params=pltpu.CompilerParams(
            dimension_semantics=("arbitrary",),
        ),
    )(x, y, m)

    return 1.0 - jnp.sum(sums) / (n * h * w)
